# baseline (speedup 1.0000x reference)
import functools

import jax
import jax.numpy as jnp
import numpy as np
from jax import lax
from jax.experimental import pallas as pl
from jax.experimental.pallas import tpu as pltpu
from jax.experimental.pallas import tpu_sc as plsc

VOCAB = 100000
D = 1024
S = 2048
B_HASH = 16
NC = 2
NS = 16
NW = NC * NS
RPW = S // NW
NCHUNK = 4
CR = RPW // NCHUNK
LANES = 16
_GSEG = 8

R2D = 784
PADV = R2D * 128

def _rvec():
    return jax.random.normal(jax.random.key(0), (B_HASH, D),
                             dtype=jnp.float32)


def _sc_body(ids_hbm, table_hbm, out_hbm, idx_v, rows_v, acc_v, *sems):
    wid = lax.axis_index("s") * NC + lax.axis_index("c")
    base = wid * RPW
    pltpu.sync_copy(ids_hbm.at[0, pl.ds(base, RPW)], idx_v)
    cps = [
        pltpu.async_copy(table_hbm.at[idx_v.at[pl.ds(k * CR, CR)]],
                         rows_v.at[pl.ds(k * CR, CR)], sems[k])
        for k in range(NCHUNK)
    ]

    zero = jnp.zeros((LANES,), jnp.float32)
    for c in range(D // LANES):
        acc_v[pl.ds(c * LANES, LANES)] = zero

    def reduce_rows(r0, nrows):
        for g in range(D // (LANES * _GSEG)):
            base_c = g * LANES * _GSEG

            def body(r, accs):
                return tuple(
                    accs[j] + rows_v[r, pl.ds(base_c + j * LANES, LANES)]
                    for j in range(_GSEG))

            init = tuple(
                acc_v[pl.ds(base_c + j * LANES, LANES)] for j in range(_GSEG))
            accs = lax.fori_loop(r0, r0 + nrows, body, init, unroll=2)
            for j in range(_GSEG):
                acc_v[pl.ds(base_c + j * LANES, LANES)] = accs[j]

    for k in range(NCHUNK):
        cps[k].wait()
        reduce_rows(k * CR, CR)
    pltpu.sync_copy(acc_v, out_hbm.at[wid])


def _make_sc_kernel():
    mesh = plsc.VectorSubcoreMesh(core_axis_name="c", subcore_axis_name="s")
    return functools.partial(
        pl.kernel,
        mesh=mesh,
        out_type=jax.ShapeDtypeStruct((NW, D), jnp.float32),
        scratch_types=[
            pltpu.VMEM((RPW,), jnp.int32),
            pltpu.VMEM((RPW, D), jnp.float32),
            pltpu.VMEM((D,), jnp.float32),
        ] + [pltpu.SemaphoreType.DMA] * NCHUNK,
    )(_sc_body)


def _lse_body(logits_ref, out_ref):
    l = logits_ref[...]
    m = jnp.max(l)
    se = jnp.sum(jnp.exp(l - m))
    out_ref[0, 0] = m
    out_ref[0, 1] = se


_lse_kernel = pl.pallas_call(
    _lse_body,
    out_shape=jax.ShapeDtypeStruct((1, 2), jnp.float32),
    in_specs=[pl.BlockSpec(memory_space=pltpu.VMEM)],
    out_specs=pl.BlockSpec(memory_space=pltpu.SMEM),
)


def _tc_body(partials_ref, logits_ref, rvec_ref, mse_ref, out_ref):
    mean = jnp.sum(partials_ref[...], axis=0, keepdims=True) * (1.0 / S)
    proj = jnp.sum(rvec_ref[...] * mean, axis=1, keepdims=True)
    bit = (proj > 0.0).astype(jnp.int32)
    row = lax.broadcasted_iota(jnp.int32, (B_HASH, 1), 0)
    weights = jnp.left_shift(jnp.int32(1), jnp.int32(B_HASH - 1) - row)
    seed = jnp.sum(bit * weights).astype(jnp.uint32)

    rr = lax.broadcasted_iota(jnp.uint32, (R2D, 128), 0)
    cc = lax.broadcasted_iota(jnp.uint32, (R2D, 128), 1)
    n = (rr << jnp.uint32(7)) | cc

    ks0 = jnp.uint32(0)
    ks1 = seed
    ks2 = seed ^ jnp.uint32(0x1BD11BDA)

    x0 = jnp.zeros((R2D, 128), jnp.uint32) + ks0
    x1 = n + ks1

    def rounds(x0, x1, rots):
        for d in rots:
            x0 = x0 + x1
            x1 = (x1 << jnp.uint32(d)) | (x1 >> jnp.uint32(32 - d))
            x1 = x0 ^ x1
        return x0, x1

    ra = (13, 15, 26, 6)
    rb = (17, 29, 16, 24)
    x0, x1 = rounds(x0, x1, ra)
    x0 = x0 + ks1
    x1 = x1 + ks2 + jnp.uint32(1)
    x0, x1 = rounds(x0, x1, rb)
    x0 = x0 + ks2
    x1 = x1 + ks0 + jnp.uint32(2)
    x0, x1 = rounds(x0, x1, ra)
    x0 = x0 + ks0
    x1 = x1 + ks1 + jnp.uint32(3)
    x0, x1 = rounds(x0, x1, rb)
    x0 = x0 + ks1
    x1 = x1 + ks2 + jnp.uint32(4)
    x0, x1 = rounds(x0, x1, ra)
    x0 = x0 + ks2
    x1 = x1 + ks0 + jnp.uint32(5)

    bits = x0 ^ x1
    fb = (bits >> jnp.uint32(9)) | jnp.uint32(0x3F800000)
    xu = lax.bitcast_convert_type(fb, jnp.float32) - 1.0

    l = logits_ref[...]
    lse = mse_ref[0, 0] + jnp.log(mse_ref[0, 1])
    ni = n.astype(jnp.int32)
    valid = ni < VOCAB
    score = jnp.where(valid, (lse - l) / xu, jnp.float32(3.0e38))

    smin = jnp.min(score)
    idx = jnp.min(jnp.where(score == smin, ni, jnp.int32(0x7FFFFFFF)))

    lane = lax.broadcasted_iota(jnp.int32, (1, VOCAB), 1)
    out_ref[...] = jnp.where(lane == idx, jnp.float32(100000.0),
                             jnp.float32(-100000.0))


_tc_kernel = pl.pallas_call(
    _tc_body,
    out_shape=jax.ShapeDtypeStruct((1, VOCAB), jnp.float32),
    in_specs=[
        pl.BlockSpec(memory_space=pltpu.VMEM),
        pl.BlockSpec(memory_space=pltpu.VMEM),
        pl.BlockSpec(memory_space=pltpu.VMEM),
        pl.BlockSpec(memory_space=pltpu.SMEM),
    ],
    out_specs=pl.BlockSpec(memory_space=pltpu.VMEM),
)


def kernel(input_ids, logits, embed_table):
    ids = input_ids.astype(jnp.int32)
    sc = _make_sc_kernel()
    partials = sc(ids, embed_table)

    lp = jnp.pad(logits, ((0, 0), (0, PADV - VOCAB)), constant_values=-1e30)
    lp2 = lp.reshape(R2D, 128)
    mse = _lse_kernel(lp2)
    return _tc_kernel(partials, lp2, _rvec(), mse)


# speedup vs baseline: 1.0626x; 1.0626x over previous
import functools

import jax
import jax.numpy as jnp
import numpy as np
from jax import lax
from jax.experimental import pallas as pl
from jax.experimental.pallas import tpu as pltpu
from jax.experimental.pallas import tpu_sc as plsc

VOCAB = 100000
D = 1024
S = 2048
B_HASH = 16
NC = 2
NS = 16
NW = NC * NS
RPW = S // NW
NCHUNK = 4
CR = RPW // NCHUNK
LANES = 16
_GSEG = 8

R2D = 784
PADV = R2D * 128

def _rvec():
    return jax.random.normal(jax.random.key(0), (B_HASH, D),
                             dtype=jnp.float32)


def _sc_body(ids_hbm, table_hbm, out_hbm, idx_v, rows_v, acc_v, *sems):
    wid = lax.axis_index("s") * NC + lax.axis_index("c")
    base = wid * RPW
    pltpu.sync_copy(ids_hbm.at[0, pl.ds(base, RPW)], idx_v)
    cps = [
        pltpu.async_copy(table_hbm.at[idx_v.at[pl.ds(k * CR, CR)]],
                         rows_v.at[pl.ds(k * CR, CR)], sems[k])
        for k in range(NCHUNK)
    ]

    def reduce_rows(r0, nrows, first):
        for g in range(D // (LANES * _GSEG)):
            base_c = g * LANES * _GSEG

            def body(r, accs):
                return tuple(
                    accs[j] + rows_v[r, pl.ds(base_c + j * LANES, LANES)]
                    for j in range(_GSEG))

            if first:
                init = tuple(
                    rows_v[r0, pl.ds(base_c + j * LANES, LANES)]
                    for j in range(_GSEG))
                lo = r0 + 1
            else:
                init = tuple(
                    acc_v[pl.ds(base_c + j * LANES, LANES)]
                    for j in range(_GSEG))
                lo = r0
            accs = lax.fori_loop(lo, r0 + nrows, body, init)
            for j in range(_GSEG):
                acc_v[pl.ds(base_c + j * LANES, LANES)] = accs[j]

    for k in range(NCHUNK):
        cps[k].wait()
        reduce_rows(k * CR, CR, k == 0)
    pltpu.sync_copy(acc_v, out_hbm.at[wid])


def _make_sc_kernel():
    mesh = plsc.VectorSubcoreMesh(core_axis_name="c", subcore_axis_name="s")
    return functools.partial(
        pl.kernel,
        mesh=mesh,
        out_type=jax.ShapeDtypeStruct((NW, D), jnp.float32),
        scratch_types=[
            pltpu.VMEM((RPW,), jnp.int32),
            pltpu.VMEM((RPW, D), jnp.float32),
            pltpu.VMEM((D,), jnp.float32),
        ] + [pltpu.SemaphoreType.DMA] * NCHUNK,
    )(_sc_body)


def _lse_body(logits_ref, out_ref):
    l = logits_ref[...]
    m = jnp.max(l)
    se = jnp.sum(jnp.exp(l - m))
    out_ref[0, 0] = m
    out_ref[0, 1] = se


_lse_kernel = pl.pallas_call(
    _lse_body,
    out_shape=jax.ShapeDtypeStruct((1, 2), jnp.float32),
    in_specs=[pl.BlockSpec(memory_space=pltpu.VMEM)],
    out_specs=pl.BlockSpec(memory_space=pltpu.SMEM),
)


def _tc_body(partials_ref, logits_ref, rvec_ref, mse_ref, out_ref):
    mean = jnp.sum(partials_ref[...], axis=0, keepdims=True) * (1.0 / S)
    proj = jnp.sum(rvec_ref[...] * mean, axis=1, keepdims=True)
    bit = (proj > 0.0).astype(jnp.int32)
    row = lax.broadcasted_iota(jnp.int32, (B_HASH, 1), 0)
    weights = jnp.left_shift(jnp.int32(1), jnp.int32(B_HASH - 1) - row)
    seed = jnp.sum(bit * weights).astype(jnp.uint32)

    rr = lax.broadcasted_iota(jnp.uint32, (R2D, 128), 0)
    cc = lax.broadcasted_iota(jnp.uint32, (R2D, 128), 1)
    n = (rr << jnp.uint32(7)) | cc

    ks0 = jnp.uint32(0)
    ks1 = seed
    ks2 = seed ^ jnp.uint32(0x1BD11BDA)

    x0 = jnp.zeros((R2D, 128), jnp.uint32) + ks0
    x1 = n + ks1

    def rounds(x0, x1, rots):
        for d in rots:
            x0 = x0 + x1
            x1 = (x1 << jnp.uint32(d)) | (x1 >> jnp.uint32(32 - d))
            x1 = x0 ^ x1
        return x0, x1

    ra = (13, 15, 26, 6)
    rb = (17, 29, 16, 24)
    x0, x1 = rounds(x0, x1, ra)
    x0 = x0 + ks1
    x1 = x1 + ks2 + jnp.uint32(1)
    x0, x1 = rounds(x0, x1, rb)
    x0 = x0 + ks2
    x1 = x1 + ks0 + jnp.uint32(2)
    x0, x1 = rounds(x0, x1, ra)
    x0 = x0 + ks0
    x1 = x1 + ks1 + jnp.uint32(3)
    x0, x1 = rounds(x0, x1, rb)
    x0 = x0 + ks1
    x1 = x1 + ks2 + jnp.uint32(4)
    x0, x1 = rounds(x0, x1, ra)
    x0 = x0 + ks2
    x1 = x1 + ks0 + jnp.uint32(5)

    bits = x0 ^ x1
    fb = (bits >> jnp.uint32(9)) | jnp.uint32(0x3F800000)
    xu = lax.bitcast_convert_type(fb, jnp.float32) - 1.0

    l = logits_ref[...]
    lse = mse_ref[0, 0] + jnp.log(mse_ref[0, 1])
    ni = n.astype(jnp.int32)
    valid = ni < VOCAB
    score = jnp.where(valid, (lse - l) / xu, jnp.float32(3.0e38))

    smin = jnp.min(score)
    idx = jnp.min(jnp.where(score == smin, ni, jnp.int32(0x7FFFFFFF)))

    lane = lax.broadcasted_iota(jnp.int32, (1, VOCAB), 1)
    out_ref[...] = jnp.where(lane == idx, jnp.float32(100000.0),
                             jnp.float32(-100000.0))


_tc_kernel = pl.pallas_call(
    _tc_body,
    out_shape=jax.ShapeDtypeStruct((1, VOCAB), jnp.float32),
    in_specs=[
        pl.BlockSpec(memory_space=pltpu.VMEM),
        pl.BlockSpec(memory_space=pltpu.VMEM),
        pl.BlockSpec(memory_space=pltpu.VMEM),
        pl.BlockSpec(memory_space=pltpu.SMEM),
    ],
    out_specs=pl.BlockSpec(memory_space=pltpu.VMEM),
)


def kernel(input_ids, logits, embed_table):
    ids = input_ids.astype(jnp.int32)
    sc = _make_sc_kernel()
    partials = sc(ids, embed_table)

    lp = jnp.pad(logits, ((0, 0), (0, PADV - VOCAB)), constant_values=-1e30)
    lp2 = lp.reshape(R2D, 128)
    mse = _lse_kernel(lp2)
    return _tc_kernel(partials, lp2, _rvec(), mse)
